# baseline (speedup 1.0000x reference)
import functools

import jax
import jax.numpy as jnp
from jax.experimental import pallas as pl
from jax.experimental.pallas import tpu as pltpu

NUM_EXPERTS = 64
NUM_GROUPS = 8
EPG = NUM_EXPERTS // NUM_GROUPS
TOP_K = 2


def _ln_group_body(x_ref, lng_ref, lnb_ref, gW1_ref, gb1_ref, gW2_ref,
                   gb2_ref, xn_out_ref, gp_out_ref, *, tb, chunks):
    f32 = jnp.float32
    bf16 = jnp.bfloat16
    c = tb // chunks
    gls = []
    for i in range(chunks):
        sl = pl.ds(i * c, c)
        xb = x_ref[sl, :]
        mu = jnp.mean(xb, axis=1, keepdims=True)
        var = jnp.mean((xb - mu) ** 2, axis=1, keepdims=True)
        xn = (xb - mu) / jnp.sqrt(var + 1e-5) * lng_ref[:, :] + lnb_ref[:, :]
        xnb = xn.astype(bf16)
        xn_out_ref[sl, :] = xnb
        h = jax.nn.relu(
            jnp.dot(xnb, gW1_ref[:, :], preferred_element_type=f32)
            + gb1_ref[:, :])
        gls.append(
            jnp.dot(h.astype(bf16), gW2_ref[:, :], preferred_element_type=f32)
            + gb2_ref[:, :])
    for i in range(chunks):
        sl = pl.ds(i * c, c)
        gl = gls[i]
        m = jnp.max(gl, axis=1, keepdims=True)
        e = jnp.exp(gl - m)
        gp_out_ref[sl, :] = e / jnp.sum(e, axis=1, keepdims=True)


def _top2_extract(pgs, base, tsb):
    iota = jax.lax.broadcasted_iota(jnp.int32, (tsb, EPG), 1) + base
    m1 = jnp.max(pgs, axis=1, keepdims=True)
    i1 = jnp.min(jnp.where(pgs == m1, iota, NUM_EXPERTS + base), axis=1,
                 keepdims=True)
    pm = jnp.where(iota == i1, -1.0, pgs)
    m2 = jnp.max(pm, axis=1, keepdims=True)
    i2 = jnp.min(jnp.where(pm == m2, iota, NUM_EXPERTS + base), axis=1,
                 keepdims=True)
    return m1, i1, m2, i2


def _merge2(a1, ai1, a2, ai2, c1, ci1, c2, ci2):
    better1 = c1 > a1
    n1 = jnp.where(better1, c1, a1)
    ni1 = jnp.where(better1, ci1, ai1)
    n2 = jnp.where(better1, jnp.where(c2 > a1, c2, a1),
                   jnp.where(c1 > a2, c1, a2))
    ni2 = jnp.where(better1, jnp.where(c2 > a1, ci2, ai1),
                    jnp.where(c1 > a2, ci1, ai2))
    return n1, ni1, n2, ni2


def _experts_body(xn_ref, gpb_ref, gpt_ref, eW1_ref, eb1_ref, eW2_ref,
                  eb2_ref, idx_out_ref, p_out_ref, aux_out_ref,
                  pg_ref, bv_ref, bi_ref, acc_ref, *, n_ts, tsb, inv_n):
    ts = pl.program_id(0)
    gs = pl.program_id(1)
    f32 = jnp.float32
    bf16 = jnp.bfloat16
    half = tsb // 2
    gprev = jax.lax.rem(gs + NUM_GROUPS - 1, NUM_GROUPS)
    first = jnp.logical_and(ts == 0, gs == 0)
    seed = gs == 1

    @pl.when(first)
    def _init_acc():
        acc_ref[:, :, :] = jnp.zeros_like(acc_ref)

    pg_prev = pg_ref[:, :]

    pg_halves = []
    for r in range(2):
        sl = pl.ds(r * half, half)
        eh = jax.nn.relu(
            jnp.dot(xn_ref[sl, :], eW1_ref[0], preferred_element_type=f32)
            + eb1_ref[0])
        el = (jnp.dot(eh.astype(bf16), eW2_ref[0],
                      preferred_element_type=f32) + eb2_ref[0])
        m = jnp.max(el, axis=1, keepdims=True)
        e = jnp.exp(el - m)
        pg_halves.append(e / jnp.sum(e, axis=1, keepdims=True))

    pgs = pg_prev * gpb_ref[0]
    c1, ci1, c2, ci2 = _top2_extract(pgs, EPG * gprev, tsb)
    a1 = jnp.where(seed, -1.0, bv_ref[:, 0:1])
    a2 = jnp.where(seed, -1.0, bv_ref[:, 1:2])
    ai1 = jnp.where(seed, NUM_EXPERTS, bi_ref[:, 0:1])
    ai2 = jnp.where(seed, NUM_EXPERTS, bi_ref[:, 1:2])
    n1, ni1, n2, ni2 = _merge2(a1, ai1, a2, ai2, c1, ci1, c2, ci2)
    bv_ref[:, :] = jnp.concatenate([n1, n2], axis=1)
    bi_ref[:, :] = jnp.concatenate([ni1, ni2], axis=1)

    colsum = jnp.where(first, 0.0, jnp.sum(pgs, axis=0, keepdims=True))
    acc_ref[gprev] = acc_ref[gprev] + colsum

    pg_ref[:, :] = jnp.concatenate(pg_halves, axis=0)

    @pl.when(jnp.logical_and(ts > 0, gs == 0))
    def _finalize_prev():
        v1, v2 = bv_ref[:, 0:1], bv_ref[:, 1:2]
        s = v1 + v2
        idx_out_ref[:, :] = bi_ref[:, :]
        p_out_ref[:, :] = jnp.concatenate([v1 / s, v2 / s], axis=1)

    @pl.when(jnp.logical_and(ts == n_ts - 1, gs == NUM_GROUPS - 1))
    def _tail():
        pgs7 = pg_ref[:, :] * gpt_ref[0]
        c1, ci1, c2, ci2 = _top2_extract(pgs7, EPG * (NUM_GROUPS - 1), tsb)
        n1, ni1, n2, ni2 = _merge2(
            bv_ref[:, 0:1], bi_ref[:, 0:1], bv_ref[:, 1:2], bi_ref[:, 1:2],
            c1, ci1, c2, ci2)
        s = n1 + n2
        idx_out_ref[:, :] = jnp.concatenate([ni1, ni2], axis=1)
        p_out_ref[:, :] = jnp.concatenate([n1 / s, n2 / s], axis=1)
        acc7 = acc_ref[NUM_GROUPS - 1] + jnp.sum(pgs7, axis=0, keepdims=True)
        cols = [acc_ref[gg] for gg in range(NUM_GROUPS - 1)] + [acc7]
        pbar = jnp.concatenate(cols, axis=1) * inv_n
        aux_out_ref[:, :] = jnp.sum(
            pbar * jnp.log(pbar * NUM_EXPERTS + 1e-9), axis=1, keepdims=True)


def kernel(x, ln_g, ln_b, gW1, gb1, gW2, gb2, eW1, eb1, eW2, eb2):
    B, S, D = x.shape
    G = NUM_GROUPS
    H2 = eW1.shape[2]
    N = B * S
    TB = 512 if N % 512 == 0 else N
    n_tb = N // TB
    TSB = 2048 if N % 2048 == 0 else N
    n_ts = N // TSB

    bf16 = jnp.bfloat16
    x2 = x.reshape(N, D)
    gW1b = gW1.astype(bf16)
    gW2b = gW2.astype(bf16)
    eW1b = eW1.astype(bf16)
    eW2b = eW2.astype(bf16)

    bodyA = functools.partial(_ln_group_body, tb=TB, chunks=4)
    xnb, gp = pl.pallas_call(
        bodyA,
        grid=(n_tb,),
        in_specs=[
            pl.BlockSpec((TB, D), lambda t: (t, 0)),
            pl.BlockSpec((1, D), lambda t: (0, 0)),
            pl.BlockSpec((1, D), lambda t: (0, 0)),
            pl.BlockSpec((D, D), lambda t: (0, 0)),
            pl.BlockSpec((1, D), lambda t: (0, 0)),
            pl.BlockSpec((D, G), lambda t: (0, 0)),
            pl.BlockSpec((1, G), lambda t: (0, 0)),
        ],
        out_specs=[
            pl.BlockSpec((TB, D), lambda t: (t, 0)),
            pl.BlockSpec((TB, G), lambda t: (t, 0)),
        ],
        out_shape=[
            jax.ShapeDtypeStruct((N, D), bf16),
            jax.ShapeDtypeStruct((N, G), jnp.float32),
        ],
        compiler_params=pltpu.CompilerParams(
            dimension_semantics=("arbitrary",)),
    )(x2, ln_g.reshape(1, D), ln_b.reshape(1, D), gW1b,
      gb1.reshape(1, D), gW2b, gb2.reshape(1, G))

    gpb = jnp.broadcast_to(jnp.swapaxes(gp, 0, 1)[:, :, None], (G, N, EPG))

    bodyB = functools.partial(_experts_body, n_ts=n_ts, tsb=TSB, inv_n=1.0 / N)

    def _gpb_idx(ts, gs):
        gprev = jax.lax.rem(gs + NUM_GROUPS - 1, NUM_GROUPS)
        row = jnp.where(gs == 0, jnp.maximum(ts - 1, 0), ts)
        return (gprev, row, 0)

    def _out_idx(ts, gs):
        return (jnp.where(gs == NUM_GROUPS - 1, ts, jnp.maximum(ts - 1, 0)), 0)

    out = pl.pallas_call(
        bodyB,
        grid=(n_ts, G),
        in_specs=[
            pl.BlockSpec((TSB, D), lambda ts, gs: (ts, 0)),
            pl.BlockSpec((1, TSB, EPG), _gpb_idx),
            pl.BlockSpec((1, TSB, EPG),
                         lambda ts, gs: (NUM_GROUPS - 1, ts, 0)),
            pl.BlockSpec((1, D, H2), lambda ts, gs: (gs, 0, 0)),
            pl.BlockSpec((1, 1, H2), lambda ts, gs: (gs, 0, 0)),
            pl.BlockSpec((1, H2, EPG), lambda ts, gs: (gs, 0, 0)),
            pl.BlockSpec((1, 1, EPG), lambda ts, gs: (gs, 0, 0)),
        ],
        out_specs=[
            pl.BlockSpec((TSB, TOP_K), _out_idx),
            pl.BlockSpec((TSB, TOP_K), _out_idx),
            pl.BlockSpec((1, 1), lambda ts, gs: (0, 0)),
        ],
        out_shape=[
            jax.ShapeDtypeStruct((N, TOP_K), jnp.int32),
            jax.ShapeDtypeStruct((N, TOP_K), jnp.float32),
            jax.ShapeDtypeStruct((1, 1), jnp.float32),
        ],
        scratch_shapes=[
            pltpu.VMEM((TSB, EPG), jnp.float32),
            pltpu.VMEM((TSB, TOP_K), jnp.float32),
            pltpu.VMEM((TSB, TOP_K), jnp.int32),
            pltpu.VMEM((G, 1, EPG), jnp.float32),
        ],
        compiler_params=pltpu.CompilerParams(
            dimension_semantics=("arbitrary", "arbitrary")),
    )(xnb, gpb, gpb, eW1b, eb1.reshape(G, 1, H2), eW2b,
      eb2.reshape(G, 1, EPG))

    top_k_indices = out[0].reshape(B, S, TOP_K)
    top_k_probs = out[1].reshape(B, S, TOP_K)
    aux_loss = out[2].reshape(())
    return (top_k_indices, top_k_probs, aux_loss)


# speedup vs baseline: 1.0809x; 1.0809x over previous
import functools

import jax
import jax.numpy as jnp
from jax.experimental import pallas as pl
from jax.experimental.pallas import tpu as pltpu

NUM_EXPERTS = 64
NUM_GROUPS = 8
EPG = NUM_EXPERTS // NUM_GROUPS
TOP_K = 2


def _ln_group_body(x_ref, lng_ref, lnb_ref, gW1_ref, gb1_ref, gW2_ref,
                   gb2_ref, xn_out_ref, gpt_out_ref, *, tb, chunks):
    f32 = jnp.float32
    bf16 = jnp.bfloat16
    c = tb // chunks
    gls = []
    for i in range(chunks):
        sl = pl.ds(i * c, c)
        xb = x_ref[sl, :]
        mu = jnp.mean(xb, axis=1, keepdims=True)
        var = jnp.mean((xb - mu) ** 2, axis=1, keepdims=True)
        xn = (xb - mu) / jnp.sqrt(var + 1e-5) * lng_ref[:, :] + lnb_ref[:, :]
        xnb = xn.astype(bf16)
        xn_out_ref[sl, :] = xnb
        h = jax.nn.relu(
            jnp.dot(xnb, gW1_ref[:, :], preferred_element_type=f32)
            + gb1_ref[:, :])
        gls.append(
            jnp.dot(h.astype(bf16), gW2_ref[:, :], preferred_element_type=f32)
            + gb2_ref[:, :])
    for i in range(chunks):
        glT = jnp.swapaxes(gls[i], 0, 1)
        m = jnp.max(glT, axis=0, keepdims=True)
        e = jnp.exp(glT - m)
        gpt_out_ref[:, pl.ds(i * c, c)] = e / jnp.sum(e, axis=0,
                                                      keepdims=True)


def _top2T_extract(pgs, base, tsb):
    iota = jax.lax.broadcasted_iota(jnp.int32, (EPG, tsb), 0) + base
    m1 = jnp.max(pgs, axis=0, keepdims=True)
    i1 = jnp.min(jnp.where(pgs == m1, iota, NUM_EXPERTS), axis=0,
                 keepdims=True)
    pm = jnp.where(iota == i1, -1.0, pgs)
    m2 = jnp.max(pm, axis=0, keepdims=True)
    i2 = jnp.min(jnp.where(pm == m2, iota, NUM_EXPERTS), axis=0,
                 keepdims=True)
    return m1, i1, m2, i2


def _merge2(a1, ai1, a2, ai2, c1, ci1, c2, ci2):
    better1 = c1 > a1
    n1 = jnp.where(better1, c1, a1)
    ni1 = jnp.where(better1, ci1, ai1)
    n2 = jnp.where(better1, jnp.where(c2 > a1, c2, a1),
                   jnp.where(c1 > a2, c1, a2))
    ni2 = jnp.where(better1, jnp.where(c2 > a1, ci2, ai1),
                    jnp.where(c1 > a2, ci1, ai2))
    return n1, ni1, n2, ni2


def _experts_body(xn_ref, gpb_ref, gpt_ref, eW1_ref, eb1_ref, eW2_ref,
                  eb2_ref, idx_out_ref, p_out_ref, aux_out_ref,
                  pg_ref, bv_ref, bi_ref, acc_ref, *, n_ts, tsb, inv_n):
    ts = pl.program_id(0)
    gs = pl.program_id(1)
    f32 = jnp.float32
    bf16 = jnp.bfloat16
    half = tsb // 2
    gprev = jax.lax.rem(gs + NUM_GROUPS - 1, NUM_GROUPS)
    first = jnp.logical_and(ts == 0, gs == 0)
    seed = gs == 1

    @pl.when(first)
    def _init_acc():
        acc_ref[:, :, :] = jnp.zeros_like(acc_ref)

    pg_prev = pg_ref[:, :]

    elTs = []
    for r in range(2):
        sl = pl.ds(r * half, half)
        eh = jax.nn.relu(
            jnp.dot(xn_ref[sl, :], eW1_ref[0], preferred_element_type=f32)
            + eb1_ref[0])
        el = (jnp.dot(eh.astype(bf16), eW2_ref[0],
                      preferred_element_type=f32) + eb2_ref[0])
        elTs.append(jnp.swapaxes(el, 0, 1))
    elT = jnp.concatenate(elTs, axis=1)
    m = jnp.max(elT, axis=0, keepdims=True)
    e = jnp.exp(elT - m)
    pg_new = e / jnp.sum(e, axis=0, keepdims=True)

    pgs = pg_prev * gpb_ref[0, 0]
    c1, ci1, c2, ci2 = _top2T_extract(pgs, EPG * gprev, tsb)
    a1 = jnp.where(seed, -1.0, bv_ref[0:1, :])
    a2 = jnp.where(seed, -1.0, bv_ref[1:2, :])
    ai1 = jnp.where(seed, NUM_EXPERTS, bi_ref[0:1, :])
    ai2 = jnp.where(seed, NUM_EXPERTS, bi_ref[1:2, :])
    n1, ni1, n2, ni2 = _merge2(a1, ai1, a2, ai2, c1, ci1, c2, ci2)
    bv_ref[:, :] = jnp.concatenate([n1, n2], axis=0)
    bi_ref[:, :] = jnp.concatenate([ni1, ni2], axis=0)

    colsum = jnp.where(first, 0.0, jnp.sum(pgs, axis=1, keepdims=True))
    acc_ref[gprev] = acc_ref[gprev] + colsum

    pg_ref[:, :] = pg_new

    @pl.when(jnp.logical_and(ts > 0, gs == 0))
    def _finalize_prev():
        v1, v2 = bv_ref[0:1, :], bv_ref[1:2, :]
        s = v1 + v2
        idx_out_ref[:, :] = bi_ref[:, :]
        p_out_ref[:, :] = jnp.concatenate([v1 / s, v2 / s], axis=0)

    @pl.when(jnp.logical_and(ts == n_ts - 1, gs == NUM_GROUPS - 1))
    def _tail():
        pgs7 = pg_ref[:, :] * gpt_ref[0, 0]
        c1, ci1, c2, ci2 = _top2T_extract(pgs7, EPG * (NUM_GROUPS - 1), tsb)
        n1, ni1, n2, ni2 = _merge2(
            bv_ref[0:1, :], bi_ref[0:1, :], bv_ref[1:2, :], bi_ref[1:2, :],
            c1, ci1, c2, ci2)
        s = n1 + n2
        idx_out_ref[:, :] = jnp.concatenate([ni1, ni2], axis=0)
        p_out_ref[:, :] = jnp.concatenate([n1 / s, n2 / s], axis=0)
        acc7 = acc_ref[NUM_GROUPS - 1] + jnp.sum(pgs7, axis=1, keepdims=True)
        cols = [acc_ref[gg] for gg in range(NUM_GROUPS - 1)] + [acc7]
        pbar = jnp.concatenate(cols, axis=0) * inv_n
        aux_out_ref[:, :] = jnp.sum(
            pbar * jnp.log(pbar * NUM_EXPERTS + 1e-9), axis=0, keepdims=True)


def kernel(x, ln_g, ln_b, gW1, gb1, gW2, gb2, eW1, eb1, eW2, eb2):
    B, S, D = x.shape
    G = NUM_GROUPS
    H2 = eW1.shape[2]
    N = B * S
    TB = 512 if N % 512 == 0 else N
    n_tb = N // TB
    TSB = 2048 if N % 2048 == 0 else N
    n_ts = N // TSB

    bf16 = jnp.bfloat16
    x2 = x.reshape(N, D)
    gW1b = gW1.astype(bf16)
    gW2b = gW2.astype(bf16)
    eW1b = eW1.astype(bf16)
    eW2b = eW2.astype(bf16)

    bodyA = functools.partial(_ln_group_body, tb=TB, chunks=4)
    xnb, gpT = pl.pallas_call(
        bodyA,
        grid=(n_tb,),
        in_specs=[
            pl.BlockSpec((TB, D), lambda t: (t, 0)),
            pl.BlockSpec((1, D), lambda t: (0, 0)),
            pl.BlockSpec((1, D), lambda t: (0, 0)),
            pl.BlockSpec((D, D), lambda t: (0, 0)),
            pl.BlockSpec((1, D), lambda t: (0, 0)),
            pl.BlockSpec((D, G), lambda t: (0, 0)),
            pl.BlockSpec((1, G), lambda t: (0, 0)),
        ],
        out_specs=[
            pl.BlockSpec((TB, D), lambda t: (t, 0)),
            pl.BlockSpec((G, TB), lambda t: (0, t)),
        ],
        out_shape=[
            jax.ShapeDtypeStruct((N, D), bf16),
            jax.ShapeDtypeStruct((G, N), jnp.float32),
        ],
        compiler_params=pltpu.CompilerParams(
            dimension_semantics=("arbitrary",)),
    )(x2, ln_g.reshape(1, D), ln_b.reshape(1, D), gW1b,
      gb1.reshape(1, D), gW2b, gb2.reshape(1, G))

    gpT4 = gpT.reshape(G, n_ts, 1, TSB)

    bodyB = functools.partial(_experts_body, n_ts=n_ts, tsb=TSB, inv_n=1.0 / N)

    def _gpb_idx(ts, gs):
        gprev = jax.lax.rem(gs + NUM_GROUPS - 1, NUM_GROUPS)
        row = jnp.where(gs == 0, jnp.maximum(ts - 1, 0), ts)
        return (gprev, row, 0, 0)

    def _out_idx(ts, gs):
        return (0, jnp.where(gs == NUM_GROUPS - 1, ts, jnp.maximum(ts - 1, 0)))

    out = pl.pallas_call(
        bodyB,
        grid=(n_ts, G),
        in_specs=[
            pl.BlockSpec((TSB, D), lambda ts, gs: (ts, 0)),
            pl.BlockSpec((1, 1, 1, TSB), _gpb_idx),
            pl.BlockSpec((1, 1, 1, TSB),
                         lambda ts, gs: (NUM_GROUPS - 1, ts, 0, 0)),
            pl.BlockSpec((1, D, H2), lambda ts, gs: (gs, 0, 0)),
            pl.BlockSpec((1, 1, H2), lambda ts, gs: (gs, 0, 0)),
            pl.BlockSpec((1, H2, EPG), lambda ts, gs: (gs, 0, 0)),
            pl.BlockSpec((1, 1, EPG), lambda ts, gs: (gs, 0, 0)),
        ],
        out_specs=[
            pl.BlockSpec((TOP_K, TSB), _out_idx),
            pl.BlockSpec((TOP_K, TSB), _out_idx),
            pl.BlockSpec((1, 1), lambda ts, gs: (0, 0)),
        ],
        out_shape=[
            jax.ShapeDtypeStruct((TOP_K, N), jnp.int32),
            jax.ShapeDtypeStruct((TOP_K, N), jnp.float32),
            jax.ShapeDtypeStruct((1, 1), jnp.float32),
        ],
        scratch_shapes=[
            pltpu.VMEM((EPG, TSB), jnp.float32),
            pltpu.VMEM((TOP_K, TSB), jnp.float32),
            pltpu.VMEM((TOP_K, TSB), jnp.int32),
            pltpu.VMEM((G, EPG, 1), jnp.float32),
        ],
        compiler_params=pltpu.CompilerParams(
            dimension_semantics=("arbitrary", "arbitrary")),
    )(xnb, gpT4, gpT4, eW1b, eb1.reshape(G, 1, H2), eW2b,
      eb2.reshape(G, 1, EPG))

    top_k_indices = jnp.swapaxes(out[0], 0, 1).reshape(B, S, TOP_K)
    top_k_probs = jnp.swapaxes(out[1], 0, 1).reshape(B, S, TOP_K)
    aux_loss = out[2].reshape(())
    return (top_k_indices, top_k_probs, aux_loss)


# speedup vs baseline: 1.0980x; 1.0159x over previous
import functools

import jax
import jax.numpy as jnp
from jax.experimental import pallas as pl
from jax.experimental.pallas import tpu as pltpu

NUM_EXPERTS = 64
NUM_GROUPS = 8
EPG = NUM_EXPERTS // NUM_GROUPS
TOP_K = 2


def _ln_group_body(x_ref, lng_ref, lnb_ref, gW1_ref, gb1_ref, gW2_ref,
                   gb2_ref, xn_out_ref, gpt_out_ref, *, tb, chunks):
    f32 = jnp.float32
    bf16 = jnp.bfloat16
    c = tb // chunks
    gls = []
    for i in range(chunks):
        sl = pl.ds(i * c, c)
        xb = x_ref[sl, :]
        mu = jnp.mean(xb, axis=1, keepdims=True)
        var = jnp.mean((xb - mu) ** 2, axis=1, keepdims=True)
        xn = (xb - mu) / jnp.sqrt(var + 1e-5) * lng_ref[:, :] + lnb_ref[:, :]
        xnb = xn.astype(bf16)
        xn_out_ref[sl, :] = xnb
        h = jax.nn.relu(
            jnp.dot(xnb, gW1_ref[:, :], preferred_element_type=f32)
            + gb1_ref[:, :])
        gls.append(
            jnp.dot(h.astype(bf16), gW2_ref[:, :], preferred_element_type=f32)
            + gb2_ref[:, :])
    for i in range(chunks):
        glT = jnp.swapaxes(gls[i], 0, 1)
        m = jnp.max(glT, axis=0, keepdims=True)
        e = jnp.exp(glT - m)
        gpt_out_ref[:, pl.ds(i * c, c)] = e / jnp.sum(e, axis=0,
                                                      keepdims=True)


def _top2T_extract(pgs, base, tsb):
    iota = jax.lax.broadcasted_iota(jnp.int32, (EPG, tsb), 0) + base
    m1 = jnp.max(pgs, axis=0, keepdims=True)
    i1 = jnp.min(jnp.where(pgs == m1, iota, NUM_EXPERTS), axis=0,
                 keepdims=True)
    pm = jnp.where(iota == i1, -1.0, pgs)
    m2 = jnp.max(pm, axis=0, keepdims=True)
    i2 = jnp.min(jnp.where(pm == m2, iota, NUM_EXPERTS), axis=0,
                 keepdims=True)
    return m1, i1, m2, i2


def _merge2(a1, ai1, a2, ai2, c1, ci1, c2, ci2):
    better1 = c1 > a1
    n1 = jnp.where(better1, c1, a1)
    ni1 = jnp.where(better1, ci1, ai1)
    n2 = jnp.where(better1, jnp.where(c2 > a1, c2, a1),
                   jnp.where(c1 > a2, c1, a2))
    ni2 = jnp.where(better1, jnp.where(c2 > a1, ci2, ai1),
                    jnp.where(c1 > a2, ci1, ai2))
    return n1, ni1, n2, ni2


def _experts_body(xn_ref, gpb_ref, gpt_ref, eW1_ref, eb1_ref, eW2_ref,
                  eb2_ref, idx_out_ref, p_out_ref, aux_out_ref,
                  pg_ref, bv_ref, bi_ref, acc_ref, *, n_ts, tsb, inv_n):
    ts = pl.program_id(0)
    gs = pl.program_id(1)
    f32 = jnp.float32
    bf16 = jnp.bfloat16
    rc = tsb // 1024 if tsb % 1024 == 0 else 2
    half = tsb // rc
    gprev = jax.lax.rem(gs + NUM_GROUPS - 1, NUM_GROUPS)
    first = jnp.logical_and(ts == 0, gs == 0)
    seed = gs == 1

    @pl.when(first)
    def _init_acc():
        acc_ref[:, :, :] = jnp.zeros_like(acc_ref)

    pg_prev = pg_ref[:, :]

    elTs = []
    for r in range(rc):
        sl = pl.ds(r * half, half)
        eh = jax.nn.relu(
            jnp.dot(xn_ref[sl, :], eW1_ref[0], preferred_element_type=f32)
            + eb1_ref[0])
        el = (jnp.dot(eh.astype(bf16), eW2_ref[0],
                      preferred_element_type=f32) + eb2_ref[0])
        elTs.append(jnp.swapaxes(el, 0, 1))
    elT = jnp.concatenate(elTs, axis=1)
    m = jnp.max(elT, axis=0, keepdims=True)
    e = jnp.exp(elT - m)
    pg_new = e / jnp.sum(e, axis=0, keepdims=True)

    pgs = pg_prev * gpb_ref[0, 0]
    c1, ci1, c2, ci2 = _top2T_extract(pgs, EPG * gprev, tsb)
    a1 = jnp.where(seed, -1.0, bv_ref[0:1, :])
    a2 = jnp.where(seed, -1.0, bv_ref[1:2, :])
    ai1 = jnp.where(seed, NUM_EXPERTS, bi_ref[0:1, :])
    ai2 = jnp.where(seed, NUM_EXPERTS, bi_ref[1:2, :])
    n1, ni1, n2, ni2 = _merge2(a1, ai1, a2, ai2, c1, ci1, c2, ci2)
    bv_ref[:, :] = jnp.concatenate([n1, n2], axis=0)
    bi_ref[:, :] = jnp.concatenate([ni1, ni2], axis=0)

    colsum = jnp.where(first, 0.0, jnp.sum(pgs, axis=1, keepdims=True))
    acc_ref[gprev] = acc_ref[gprev] + colsum

    pg_ref[:, :] = pg_new

    @pl.when(jnp.logical_and(ts > 0, gs == 0))
    def _finalize_prev():
        v1, v2 = bv_ref[0:1, :], bv_ref[1:2, :]
        s = v1 + v2
        idx_out_ref[:, :] = bi_ref[:, :]
        p_out_ref[:, :] = jnp.concatenate([v1 / s, v2 / s], axis=0)

    @pl.when(jnp.logical_and(ts == n_ts - 1, gs == NUM_GROUPS - 1))
    def _tail():
        pgs7 = pg_ref[:, :] * gpt_ref[0, 0]
        c1, ci1, c2, ci2 = _top2T_extract(pgs7, EPG * (NUM_GROUPS - 1), tsb)
        n1, ni1, n2, ni2 = _merge2(
            bv_ref[0:1, :], bi_ref[0:1, :], bv_ref[1:2, :], bi_ref[1:2, :],
            c1, ci1, c2, ci2)
        s = n1 + n2
        idx_out_ref[:, :] = jnp.concatenate([ni1, ni2], axis=0)
        p_out_ref[:, :] = jnp.concatenate([n1 / s, n2 / s], axis=0)
        acc7 = acc_ref[NUM_GROUPS - 1] + jnp.sum(pgs7, axis=1, keepdims=True)
        cols = [acc_ref[gg] for gg in range(NUM_GROUPS - 1)] + [acc7]
        pbar = jnp.concatenate(cols, axis=0) * inv_n
        aux_out_ref[:, :] = jnp.sum(
            pbar * jnp.log(pbar * NUM_EXPERTS + 1e-9), axis=0, keepdims=True)


def kernel(x, ln_g, ln_b, gW1, gb1, gW2, gb2, eW1, eb1, eW2, eb2):
    B, S, D = x.shape
    G = NUM_GROUPS
    H2 = eW1.shape[2]
    N = B * S
    TB = 1024 if N % 1024 == 0 else N
    n_tb = N // TB
    TSB = 4096 if N % 4096 == 0 else (2048 if N % 2048 == 0 else N)
    n_ts = N // TSB

    bf16 = jnp.bfloat16
    x2 = x.reshape(N, D)
    gW1b = gW1.astype(bf16)
    gW2b = gW2.astype(bf16)
    eW1b = eW1.astype(bf16)
    eW2b = eW2.astype(bf16)

    bodyA = functools.partial(_ln_group_body, tb=TB, chunks=4)
    xnb, gpT = pl.pallas_call(
        bodyA,
        grid=(n_tb,),
        in_specs=[
            pl.BlockSpec((TB, D), lambda t: (t, 0)),
            pl.BlockSpec((1, D), lambda t: (0, 0)),
            pl.BlockSpec((1, D), lambda t: (0, 0)),
            pl.BlockSpec((D, D), lambda t: (0, 0)),
            pl.BlockSpec((1, D), lambda t: (0, 0)),
            pl.BlockSpec((D, G), lambda t: (0, 0)),
            pl.BlockSpec((1, G), lambda t: (0, 0)),
        ],
        out_specs=[
            pl.BlockSpec((TB, D), lambda t: (t, 0)),
            pl.BlockSpec((G, TB), lambda t: (0, t)),
        ],
        out_shape=[
            jax.ShapeDtypeStruct((N, D), bf16),
            jax.ShapeDtypeStruct((G, N), jnp.float32),
        ],
        compiler_params=pltpu.CompilerParams(
            dimension_semantics=("arbitrary",)),
    )(x2, ln_g.reshape(1, D), ln_b.reshape(1, D), gW1b,
      gb1.reshape(1, D), gW2b, gb2.reshape(1, G))

    gpT4 = gpT.reshape(G, n_ts, 1, TSB)

    bodyB = functools.partial(_experts_body, n_ts=n_ts, tsb=TSB, inv_n=1.0 / N)

    def _gpb_idx(ts, gs):
        gprev = jax.lax.rem(gs + NUM_GROUPS - 1, NUM_GROUPS)
        row = jnp.where(gs == 0, jnp.maximum(ts - 1, 0), ts)
        return (gprev, row, 0, 0)

    def _out_idx(ts, gs):
        return (0, jnp.where(gs == NUM_GROUPS - 1, ts, jnp.maximum(ts - 1, 0)))

    out = pl.pallas_call(
        bodyB,
        grid=(n_ts, G),
        in_specs=[
            pl.BlockSpec((TSB, D), lambda ts, gs: (ts, 0)),
            pl.BlockSpec((1, 1, 1, TSB), _gpb_idx),
            pl.BlockSpec((1, 1, 1, TSB),
                         lambda ts, gs: (NUM_GROUPS - 1, ts, 0, 0)),
            pl.BlockSpec((1, D, H2), lambda ts, gs: (gs, 0, 0)),
            pl.BlockSpec((1, 1, H2), lambda ts, gs: (gs, 0, 0)),
            pl.BlockSpec((1, H2, EPG), lambda ts, gs: (gs, 0, 0)),
            pl.BlockSpec((1, 1, EPG), lambda ts, gs: (gs, 0, 0)),
        ],
        out_specs=[
            pl.BlockSpec((TOP_K, TSB), _out_idx),
            pl.BlockSpec((TOP_K, TSB), _out_idx),
            pl.BlockSpec((1, 1), lambda ts, gs: (0, 0)),
        ],
        out_shape=[
            jax.ShapeDtypeStruct((TOP_K, N), jnp.int32),
            jax.ShapeDtypeStruct((TOP_K, N), jnp.float32),
            jax.ShapeDtypeStruct((1, 1), jnp.float32),
        ],
        scratch_shapes=[
            pltpu.VMEM((EPG, TSB), jnp.float32),
            pltpu.VMEM((TOP_K, TSB), jnp.float32),
            pltpu.VMEM((TOP_K, TSB), jnp.int32),
            pltpu.VMEM((G, EPG, 1), jnp.float32),
        ],
        compiler_params=pltpu.CompilerParams(
            dimension_semantics=("arbitrary", "arbitrary")),
    )(xnb, gpT4, gpT4, eW1b, eb1.reshape(G, 1, H2), eW2b,
      eb2.reshape(G, 1, EPG))

    top_k_indices = jnp.swapaxes(out[0], 0, 1).reshape(B, S, TOP_K)
    top_k_probs = jnp.swapaxes(out[1], 0, 1).reshape(B, S, TOP_K)
    aux_loss = out[2].reshape(())
    return (top_k_indices, top_k_probs, aux_loss)
